# baseline (speedup 1.0000x reference)
import functools

import jax
import jax.numpy as jnp
from jax import lax
from jax.experimental import pallas as pl
from jax.experimental.pallas import tpu as pltpu
from jax.experimental.pallas import tpu_sc as plsc

F = 32
L = 16
CB = 16
GR = 80
TBW = 512


def _tp_body(x_ref, y_ref):
    y_ref[...] = jax.lax.dot_general(
        x_ref[...], jnp.eye(F, dtype=jnp.float32),
        (((0,), (0,)), ((), ())),
        preferred_element_type=jnp.float32,
        precision=jax.lax.Precision.HIGHEST)


@functools.cache
def _transpose(N):
    return pl.pallas_call(
        _tp_body,
        grid=(pl.cdiv(N, TBW),),
        in_specs=[pl.BlockSpec((F, TBW), lambda j: (0, j))],
        out_specs=pl.BlockSpec((TBW, F), lambda j: (j, 0)),
        out_shape=jax.ShapeDtypeStruct((N, F), jnp.float32),
    )


@functools.cache
def _build(B, H):
    info = plsc.get_sparse_core_info()
    NC, NS = info.num_cores, info.num_subcores
    NW = NC * NS
    assert B % (NW * CB) == 0
    BPW = B // NW
    n_chunks = BPW // CB

    mesh = plsc.VectorSubcoreMesh(core_axis_name="c", subcore_axis_name="s")

    @functools.partial(
        pl.kernel,
        mesh=mesh,
        compiler_params=pltpu.CompilerParams(
            needs_layout_passes=False, use_tc_tiling_on_sc=False),
        out_type=jax.ShapeDtypeStruct((B,), jnp.float32),
        scratch_types=[
            pltpu.VMEM((CB,), jnp.int32),
            pltpu.VMEM((CB,), jnp.int32),
            pltpu.VMEM((CB * H,), jnp.int32),
            pltpu.VMEM((CB, F), jnp.float32),
            pltpu.VMEM((CB, F), jnp.float32),
            pltpu.VMEM((CB * H, F), jnp.float32),
            pltpu.VMEM((CB, L), jnp.float32),
            pltpu.VMEM((BPW,), jnp.float32),
            pltpu.SemaphoreType.DMA,
        ],
    )
    def _k(th_h, be_h, rh_h, us_h, it_h, cx_h, out_h,
           uidx, iidx, cidx, th_v, be_v, rho_v, q_v, outb, sem):
        wid = lax.axis_index("s") * NC + lax.axis_index("c")
        base = pl.multiple_of(wid * BPW, BPW)

        def chunk_body(c, carry):
            gb = pl.multiple_of(base + c * CB, CB)
            gbc = pl.multiple_of((base + c * CB) * H, CB * H)
            pltpu.sync_copy(us_h.at[pl.ds(gb, CB)], uidx)
            pltpu.sync_copy(it_h.at[pl.ds(gb, CB)], iidx)
            pltpu.sync_copy(cx_h.at[pl.ds(gbc, CB * H)], cidx)
            cps = [
                pltpu.async_copy(th_h.at[uidx], th_v, sem),
                pltpu.async_copy(be_h.at[iidx], be_v, sem),
            ]
            for g in range(0, CB * H, GR):
                cps.append(pltpu.async_copy(
                    rh_h.at[cidx.at[pl.ds(g, GR)]],
                    rho_v.at[pl.ds(g, GR)], sem))
            for cp in cps:
                cp.wait()

            def e_body(e, carry2):
                acc0 = th_v[e, pl.ds(0, L)]
                acc1 = th_v[e, pl.ds(L, L)]
                for h in range(H):
                    acc0 = acc0 + rho_v[e * H + h, pl.ds(0, L)]
                    acc1 = acc1 + rho_v[e * H + h, pl.ds(L, L)]
                q_v[e, pl.ds(0, L)] = (be_v[e, pl.ds(0, L)] * acc0
                                       + be_v[e, pl.ds(L, L)] * acc1)
                return carry2

            lax.fori_loop(0, CB, e_body, 0)

            lanes = lax.iota(jnp.int32, L)
            svec = jnp.zeros((L,), jnp.float32)
            for j in range(L):
                svec = svec + plsc.load_gather(
                    q_v, [lanes, jnp.full((L,), j, jnp.int32)])
            outb[pl.ds(pl.multiple_of(c * CB, CB), CB)] = svec
            return carry

        lax.fori_loop(0, n_chunks, chunk_body, 0)

        def sig_body(i, carry):
            off = pl.multiple_of(i * L, L)
            x = outb[pl.ds(off, L)]
            outb[pl.ds(off, L)] = 1.0 / (1.0 + jnp.exp(-x))
            return carry

        lax.fori_loop(0, BPW // L, sig_body, 0)
        pltpu.sync_copy(outb, out_h.at[pl.ds(base, BPW)])

    return _k


def kernel(theta, beta, rho, user, item, contexts):
    B, H = contexts.shape
    N = theta.shape[0]
    tp = _transpose(N)
    return _build(B, H)(tp(theta.T), tp(beta.T), tp(rho.T),
                        user, item, contexts.reshape(B * H))


# speedup vs baseline: 3.9518x; 3.9518x over previous
import functools

import jax
import jax.numpy as jnp
from jax import lax
from jax.experimental import pallas as pl
from jax.experimental.pallas import tpu as pltpu
from jax.experimental.pallas import tpu_sc as plsc

F = 32
L = 16
CB = 16
GR = 80
UG = 128


@functools.cache
def _build(B, H):
    info = plsc.get_sparse_core_info()
    NC, NS = info.num_cores, info.num_subcores
    NW = NC * NS
    assert B % (NW * CB) == 0
    BPW = B // NW
    n_chunks = BPW // CB
    CR = CB * H

    mesh = plsc.VectorSubcoreMesh(core_axis_name="c", subcore_axis_name="s")

    @functools.partial(
        pl.kernel,
        mesh=mesh,
        compiler_params=pltpu.CompilerParams(
            needs_layout_passes=False, use_tc_tiling_on_sc=False),
        out_type=jax.ShapeDtypeStruct((B,), jnp.float32),
        scratch_types=[
            pltpu.VMEM((BPW,), jnp.int32),
            pltpu.VMEM((BPW,), jnp.int32),
            pltpu.VMEM((BPW, F), jnp.float32),
            pltpu.VMEM((BPW, F), jnp.float32),
            pltpu.VMEM((CR,), jnp.int32),
            pltpu.VMEM((CR,), jnp.int32),
            pltpu.VMEM((CR, F), jnp.float32),
            pltpu.VMEM((CR, F), jnp.float32),
            pltpu.VMEM((CB, L), jnp.float32),
            pltpu.VMEM((BPW,), jnp.float32),
            pltpu.SemaphoreType.DMA,
            pltpu.SemaphoreType.DMA,
            pltpu.SemaphoreType.DMA,
        ],
    )
    def _k(th_h, be_h, rh_h, us_h, it_h, cx_h, out_h,
           uidx, iidx, th_v, be_v, cidx0, cidx1, rho0, rho1,
           q_v, outb, sem0, sem1, sem2):
        wid = lax.axis_index("s") * NC + lax.axis_index("c")
        base = pl.multiple_of(wid * BPW, BPW)

        pltpu.sync_copy(us_h.at[pl.ds(base, BPW)], uidx)
        pltpu.sync_copy(it_h.at[pl.ds(base, BPW)], iidx)
        for g in range(0, BPW, UG):
            pltpu.async_copy(th_h.at[uidx.at[pl.ds(g, UG)]],
                             th_v.at[pl.ds(g, UG)], sem2)
            pltpu.async_copy(be_h.at[iidx.at[pl.ds(g, UG)]],
                             be_v.at[pl.ds(g, UG)], sem2)

        def fire(c, cidx, rho, sem):
            gbc = pl.multiple_of((base + c * CB) * H, CR)
            pltpu.sync_copy(cx_h.at[pl.ds(gbc, CR)], cidx)
            for g in range(0, CR, GR):
                pltpu.async_copy(rh_h.at[cidx.at[pl.ds(g, GR)]],
                                 rho.at[pl.ds(g, GR)], sem)

        def drain(cidx, rho, sem):
            for g in range(0, CR, GR):
                pltpu.make_async_copy(rh_h.at[cidx.at[pl.ds(g, GR)]],
                                      rho.at[pl.ds(g, GR)], sem).wait()

        def compute(c, rho):
            def e_body(e, carry):
                acc0 = th_v[c * CB + e, pl.ds(0, L)]
                acc1 = th_v[c * CB + e, pl.ds(L, L)]
                for h in range(H):
                    acc0 = acc0 + rho[e * H + h, pl.ds(0, L)]
                    acc1 = acc1 + rho[e * H + h, pl.ds(L, L)]
                q_v[e, pl.ds(0, L)] = (
                    be_v[c * CB + e, pl.ds(0, L)] * acc0
                    + be_v[c * CB + e, pl.ds(L, L)] * acc1)
                return carry

            lax.fori_loop(0, CB, e_body, 0)

            lanes = lax.iota(jnp.int32, L)
            svec = jnp.zeros((L,), jnp.float32)
            for j in range(L):
                svec = svec + plsc.load_gather(
                    q_v, [lanes, jnp.full((L,), j, jnp.int32)])
            outb[pl.ds(pl.multiple_of(c * CB, CB), CB)] = svec

        fire(0, cidx0, rho0, sem0)

        for g in range(0, BPW, UG):
            pltpu.make_async_copy(th_h.at[uidx.at[pl.ds(g, UG)]],
                                  th_v.at[pl.ds(g, UG)], sem2).wait()
            pltpu.make_async_copy(be_h.at[iidx.at[pl.ds(g, UG)]],
                                  be_v.at[pl.ds(g, UG)], sem2).wait()

        def pair_body(p, carry):
            c0 = p * 2
            pl.when(c0 + 1 < n_chunks)(
                lambda: fire(c0 + 1, cidx1, rho1, sem1))
            drain(cidx0, rho0, sem0)
            compute(c0, rho0)
            pl.when(c0 + 2 < n_chunks)(
                lambda: fire(c0 + 2, cidx0, rho0, sem0))
            drain(cidx1, rho1, sem1)
            compute(c0 + 1, rho1)
            return carry

        lax.fori_loop(0, n_chunks // 2, pair_body, 0)

        def sig_body(i, carry):
            off = pl.multiple_of(i * L, L)
            x = outb[pl.ds(off, L)]
            outb[pl.ds(off, L)] = 1.0 / (1.0 + jnp.exp(-x))
            return carry

        lax.fori_loop(0, BPW // L, sig_body, 0)
        pltpu.sync_copy(outb, out_h.at[pl.ds(base, BPW)])

    return _k


def kernel(theta, beta, rho, user, item, contexts):
    B, H = contexts.shape
    return _build(B, H)(theta, beta, rho, user, item,
                        contexts.reshape(B * H))


# speedup vs baseline: 4.0807x; 1.0326x over previous
import functools

import jax
import jax.numpy as jnp
from jax import lax
from jax.experimental import pallas as pl
from jax.experimental.pallas import tpu as pltpu
from jax.experimental.pallas import tpu_sc as plsc

F = 32
L = 16
CB = 16
GR = 80
UG = 128

_MESH = None


def _mesh():
    global _MESH
    if _MESH is None:
        _MESH = plsc.VectorSubcoreMesh(core_axis_name="c", subcore_axis_name="s")
    return _MESH


_PARAMS = None


def _params():
    global _PARAMS
    if _PARAMS is None:
        _PARAMS = pltpu.CompilerParams(
            needs_layout_passes=False, use_tc_tiling_on_sc=False)
    return _PARAMS


@functools.cache
def _build_rsum(B, H):
    info = plsc.get_sparse_core_info()
    NC, NS = info.num_cores, info.num_subcores
    NW = NC * NS
    BPW = B // NW
    n_chunks = BPW // CB
    CR = CB * H

    @functools.partial(
        pl.kernel,
        mesh=_mesh(),
        compiler_params=_params(),
        out_type=jax.ShapeDtypeStruct((B, F), jnp.float32),
        scratch_types=[
            pltpu.VMEM((CR,), jnp.int32),
            pltpu.VMEM((CR,), jnp.int32),
            pltpu.VMEM((CR, F), jnp.float32),
            pltpu.VMEM((CR, F), jnp.float32),
            pltpu.VMEM((BPW, F), jnp.float32),
            pltpu.SemaphoreType.DMA,
            pltpu.SemaphoreType.DMA,
        ],
    )
    def _k(rh_h, cx_h, out_h, cidx0, cidx1, rho0, rho1, rs_v, sem0, sem1):
        wid = lax.axis_index("s") * NC + lax.axis_index("c")
        base = pl.multiple_of(wid * BPW, BPW)

        def fire(c, cidx, rho, sem):
            gbc = pl.multiple_of((base + c * CB) * H, CR)
            pltpu.sync_copy(cx_h.at[pl.ds(gbc, CR)], cidx)
            for g in range(0, CR, GR):
                pltpu.async_copy(rh_h.at[cidx.at[pl.ds(g, GR)]],
                                 rho.at[pl.ds(g, GR)], sem)

        def drain(cidx, rho, sem):
            for g in range(0, CR, GR):
                pltpu.make_async_copy(rh_h.at[cidx.at[pl.ds(g, GR)]],
                                      rho.at[pl.ds(g, GR)], sem).wait()

        def compute(c, rho):
            def e_body(e, carry):
                acc0 = rho[e * H, pl.ds(0, L)]
                acc1 = rho[e * H, pl.ds(L, L)]
                for h in range(1, H):
                    acc0 = acc0 + rho[e * H + h, pl.ds(0, L)]
                    acc1 = acc1 + rho[e * H + h, pl.ds(L, L)]
                rs_v[c * CB + e, pl.ds(0, L)] = acc0
                rs_v[c * CB + e, pl.ds(L, L)] = acc1
                return carry

            lax.fori_loop(0, CB, e_body, 0)

        fire(0, cidx0, rho0, sem0)

        def pair_body(p, carry):
            c0 = p * 2
            pl.when(c0 + 1 < n_chunks)(
                lambda: fire(c0 + 1, cidx1, rho1, sem1))
            drain(cidx0, rho0, sem0)
            compute(c0, rho0)
            pl.when(c0 + 2 < n_chunks)(
                lambda: fire(c0 + 2, cidx0, rho0, sem0))
            drain(cidx1, rho1, sem1)
            compute(c0 + 1, rho1)
            return carry

        lax.fori_loop(0, n_chunks // 2, pair_body, 0)
        pltpu.sync_copy(rs_v, out_h.at[pl.ds(base, BPW)])

    return _k


@functools.cache
def _build_logit(B):
    info = plsc.get_sparse_core_info()
    NC, NS = info.num_cores, info.num_subcores
    NW = NC * NS
    BPW = B // NW
    n_chunks = BPW // CB

    @functools.partial(
        pl.kernel,
        mesh=_mesh(),
        compiler_params=_params(),
        out_type=jax.ShapeDtypeStruct((B,), jnp.float32),
        scratch_types=[
            pltpu.VMEM((BPW,), jnp.int32),
            pltpu.VMEM((BPW,), jnp.int32),
            pltpu.VMEM((BPW, F), jnp.float32),
            pltpu.VMEM((BPW, F), jnp.float32),
            pltpu.VMEM((BPW, F), jnp.float32),
            pltpu.VMEM((CB, L), jnp.float32),
            pltpu.VMEM((BPW,), jnp.float32),
            pltpu.SemaphoreType.DMA,
        ],
    )
    def _k(th_h, be_h, us_h, it_h, rs_h, out_h,
           uidx, iidx, th_v, be_v, rs_v, q_v, outb, sem):
        wid = lax.axis_index("s") * NC + lax.axis_index("c")
        base = pl.multiple_of(wid * BPW, BPW)

        pltpu.sync_copy(us_h.at[pl.ds(base, BPW)], uidx)
        pltpu.sync_copy(it_h.at[pl.ds(base, BPW)], iidx)
        cps = [pltpu.async_copy(rs_h.at[pl.ds(base, BPW)], rs_v, sem)]
        for g in range(0, BPW, UG):
            cps.append(pltpu.async_copy(th_h.at[uidx.at[pl.ds(g, UG)]],
                                        th_v.at[pl.ds(g, UG)], sem))
            cps.append(pltpu.async_copy(be_h.at[iidx.at[pl.ds(g, UG)]],
                                        be_v.at[pl.ds(g, UG)], sem))
        for cp in cps:
            cp.wait()

        lanes = lax.iota(jnp.int32, L)

        def chunk_body(c, carry):
            def e_body(e, carry2):
                ge = c * CB + e
                acc0 = th_v[ge, pl.ds(0, L)] + rs_v[ge, pl.ds(0, L)]
                acc1 = th_v[ge, pl.ds(L, L)] + rs_v[ge, pl.ds(L, L)]
                q_v[e, pl.ds(0, L)] = (be_v[ge, pl.ds(0, L)] * acc0
                                       + be_v[ge, pl.ds(L, L)] * acc1)
                return carry2

            lax.fori_loop(0, CB, e_body, 0)
            svec = jnp.zeros((L,), jnp.float32)
            for j in range(L):
                svec = svec + plsc.load_gather(
                    q_v, [lanes, jnp.full((L,), j, jnp.int32)])
            svec = 1.0 / (1.0 + jnp.exp(-svec))
            outb[pl.ds(pl.multiple_of(c * CB, CB), CB)] = svec
            return carry

        lax.fori_loop(0, n_chunks, chunk_body, 0)
        pltpu.sync_copy(outb, out_h.at[pl.ds(base, BPW)])

    return _k


def kernel(theta, beta, rho, user, item, contexts):
    B, H = contexts.shape
    rsum = _build_rsum(B, H)(rho, contexts.reshape(B * H))
    return _build_logit(B)(theta, beta, user, item, rsum)
